# baseline (speedup 1.0000x reference)
import functools

import jax
import jax.numpy as jnp
from jax import lax
from jax.experimental import pallas as pl
from jax.experimental.pallas import tpu as pltpu
from jax.experimental.pallas import tpu_sc as plsc

N = 10000
D = 128
DH = D // 2
R = 8
NB = 4
BS = D // NB
E2 = 2 * 320000

NC = 2
NS = 16
L = 16
K = 128
PH = 4
NSLOT = 2
CHP = 80
EP = NS * PH * CHP * K
NP = 10240
ROWS_PER_TILE = NP // NS


def _mm_body(x_ref, w_ref, o_ref):
    o_ref[...] = jnp.dot(x_ref[...], w_ref[...],
                         preferred_element_type=jnp.float32)


def _interleave_body(a_ref, b_ref, o_ref):
    o_ref[:, :DH] = a_ref[...]
    o_ref[:, DH:] = b_ref[...]


def _sc_body(xr_hbm, gidx_hbm, tgt_hbm, w_hbm, zeros_hbm, out_hbm,
             gidx_v, tgt_v, w_v, rows_v, acc_sh, gsem, stsem):
    cid = lax.axis_index("c")
    sid = lax.axis_index("s")

    def stage_start(ph, b):
        pltpu.async_copy(gidx_hbm.at[cid].at[sid].at[ph], gidx_v.at[b], stsem)
        pltpu.async_copy(tgt_hbm.at[sid].at[ph], tgt_v.at[b], stsem)
        pltpu.async_copy(w_hbm.at[sid].at[ph], w_v.at[b], stsem)

    def stage_wait(ph, b):
        pltpu.make_async_copy(gidx_hbm.at[cid].at[sid].at[ph], gidx_v.at[b],
                              stsem).wait()
        pltpu.make_async_copy(tgt_hbm.at[sid].at[ph], tgt_v.at[b],
                              stsem).wait()
        pltpu.make_async_copy(w_hbm.at[sid].at[ph], w_v.at[b], stsem).wait()

    def gather_start(b, c, s):
        pltpu.async_copy(xr_hbm.at[gidx_v.at[b].at[c]], rows_v.at[s],
                         gsem.at[s])

    def gather_wait(b, c, s):
        pltpu.make_async_copy(xr_hbm.at[gidx_v.at[b].at[c]], rows_v.at[s],
                              gsem.at[s]).wait()

    def scatter_add(b, c, s):
        pltpu.sync_copy(rows_v.at[s], acc_sh.at[tgt_v.at[b].at[c]], add=True)

    def scale(b, c, s):
        def grp_body(g, carry):
            wv = w_v[b, c, pl.ds(g * L, L)]
            for k in range(L):
                wk = wv[k]
                row = g * L + k
                for f in range(DH // L):
                    fs = pl.ds(f * L, L)
                    rows_v[s, row, fs] = rows_v[s, row, fs] * wk
            return carry

        lax.fori_loop(0, K // L, grp_body, 0)

    stage_start(0, 0)
    sl = pl.ds(sid * ROWS_PER_TILE, ROWS_PER_TILE)
    pltpu.sync_copy(zeros_hbm.at[sl], acc_sh.at[sl])
    plsc.subcore_barrier()

    for ph in range(PH):
        b = ph % 2
        stage_wait(ph, b)
        if ph + 1 < PH:
            stage_start(ph + 1, 1 - b)

        for c in range(NSLOT - 1):
            gather_start(b, c, c)

        def ring_body(g, carry):
            for s in range(NSLOT):
                c = g * NSLOT + s
                ahead = c + NSLOT - 1
                sa = (s + NSLOT - 1) % NSLOT

                @pl.when(ahead < CHP)
                def _():
                    gather_start(b, ahead, sa)

                gather_wait(b, c, s)
                scale(b, c, s)
                scatter_add(b, c, s)
            return carry

        lax.fori_loop(0, CHP // NSLOT, ring_body, 0)

    plsc.subcore_barrier()

    pltpu.sync_copy(acc_sh.at[sl], out_hbm.at[cid].at[sl])


def kernel(x, source, target, edge_type, edge_weights, blocks):
    w8 = jnp.zeros((R, D, D), jnp.float32)
    for b in range(NB):
        s = b * BS
        w8 = w8.at[:, s:s + BS, s:s + BS].set(blocks[:R, b])
    wcat = jnp.transpose(w8, (1, 0, 2)).reshape(D, R * D)

    TN = 1000
    xr = pl.pallas_call(
        _mm_body,
        grid=(N // TN,),
        in_specs=[
            pl.BlockSpec((TN, D), lambda i: (i, 0)),
            pl.BlockSpec((D, R * D), lambda i: (0, 0)),
        ],
        out_specs=pl.BlockSpec((TN, R * D), lambda i: (i, 0)),
        out_shape=jax.ShapeDtypeStruct((N, R * D), jnp.float32),
    )(x, wcat)
    xr = xr.reshape(N * R * 2, DH)

    src_all = jnp.concatenate([source, target])
    tgt_all = jnp.concatenate([target, source])
    et_all = jnp.concatenate([edge_type, edge_type])
    w_all = jnp.concatenate([edge_weights, edge_weights])
    g2 = ((src_all * R + et_all) * 2).astype(jnp.int32)

    pad = EP - E2
    shp = (NS, PH, CHP, K)
    gidx5 = jnp.pad(jnp.stack([g2, g2 + 1]),
                    ((0, 0), (0, pad))).reshape((NC,) + shp)
    tgt4 = jnp.pad(tgt_all.astype(jnp.int32), (0, pad)).reshape(shp)
    w4 = jnp.pad(w_all, (0, pad)).reshape(shp)
    zeros = jnp.zeros((NP, DH), jnp.float32)

    mesh = plsc.VectorSubcoreMesh(core_axis_name="c", subcore_axis_name="s")
    sc_kernel = functools.partial(
        pl.kernel,
        mesh=mesh,
        compiler_params=pltpu.CompilerParams(use_tc_tiling_on_sc=False),
        out_type=jax.ShapeDtypeStruct((NC, NP, DH), jnp.float32),
        scratch_types=[
            pltpu.VMEM((2, CHP, K), jnp.int32),
            pltpu.VMEM((2, CHP, K), jnp.int32),
            pltpu.VMEM((2, CHP, K), jnp.float32),
            pltpu.VMEM((NSLOT, K, DH), jnp.float32),
            pltpu.VMEM_SHARED((NP, DH), jnp.float32),
            pltpu.SemaphoreType.DMA((NSLOT,)),
            pltpu.SemaphoreType.DMA,
        ],
    )(_sc_body)
    partials = sc_kernel(xr, gidx5, tgt4, w4, zeros)

    out = pl.pallas_call(
        _interleave_body,
        grid=(N // TN,),
        in_specs=[
            pl.BlockSpec((TN, DH), lambda i: (i, 0)),
            pl.BlockSpec((TN, DH), lambda i: (i, 0)),
        ],
        out_specs=pl.BlockSpec((TN, D), lambda i: (i, 0)),
        out_shape=jax.ShapeDtypeStruct((N, D), jnp.float32),
    )(partials[0, :N], partials[1, :N])
    return out


# speedup vs baseline: 1.1805x; 1.1805x over previous
import functools

import jax
import jax.numpy as jnp
from jax import lax
from jax.experimental import pallas as pl
from jax.experimental.pallas import tpu as pltpu
from jax.experimental.pallas import tpu_sc as plsc

N = 10000
D = 128
DH = D // 2
R = 8
NB = 4
BS = D // NB
E2 = 2 * 320000

NC = 2
NS = 16
L = 16
K = 128
PH = 2
NSLOT = 2
CHP = 158
EP = NS * PH * CHP * K
NP = 10240
ROWS_PER_TILE = NP // NS


def _mm_body(x_ref, w_ref, o_ref):
    o_ref[...] = jnp.dot(x_ref[...], w_ref[...],
                         preferred_element_type=jnp.float32)


def _interleave_body(a_ref, b_ref, o_ref):
    o_ref[:, :DH] = a_ref[...]
    o_ref[:, DH:] = b_ref[...]


def _sc_body(xr_hbm, gidx_hbm, tgt_hbm, w_hbm, zeros_hbm, out_hbm,
             gidx_v, tgt_v, w_v, rows_a, rows_b, acc_sh, gsem):
    cid = lax.axis_index("c")
    sid = lax.axis_index("s")
    bufs = (rows_a, rows_b)

    def gather_start(c, s):
        pltpu.async_copy(xr_hbm.at[gidx_v.at[c]], bufs[s], gsem.at[s])

    def gather_wait(c, s):
        pltpu.make_async_copy(xr_hbm.at[gidx_v.at[c]], bufs[s],
                              gsem.at[s]).wait()

    def scatter_add(c, s):
        pltpu.sync_copy(bufs[s], acc_sh.at[tgt_v.at[c]], add=True)

    def scale(c, s):
        rv = bufs[s]

        def grp_body(g, carry):
            wv = w_v[c, pl.ds(g * L, L)]
            for k in range(L):
                wk = wv[k]
                row = g * L + k
                for f in range(DH // L):
                    fs = pl.ds(f * L, L)
                    rv[row, fs] = rv[row, fs] * wk
            return carry

        lax.fori_loop(0, K // L, grp_body, 0)

    sl = pl.ds(sid * ROWS_PER_TILE, ROWS_PER_TILE)
    pltpu.sync_copy(zeros_hbm.at[sl], acc_sh.at[sl])
    plsc.subcore_barrier()

    for ph in range(PH):
        pltpu.sync_copy(gidx_hbm.at[cid].at[sid].at[ph], gidx_v)
        pltpu.sync_copy(tgt_hbm.at[sid].at[ph], tgt_v)
        pltpu.sync_copy(w_hbm.at[sid].at[ph], w_v)

        gather_start(0, 0)

        def pair_body(g, carry):
            c0 = 2 * g
            gather_start(c0 + 1, 1)
            gather_wait(c0, 0)
            scale(c0, 0)
            scatter_add(c0, 0)

            @pl.when(c0 + 2 < CHP)
            def _():
                gather_start(c0 + 2, 0)

            gather_wait(c0 + 1, 1)
            scale(c0 + 1, 1)
            scatter_add(c0 + 1, 1)
            return carry

        lax.fori_loop(0, CHP // 2, pair_body, 0)

    plsc.subcore_barrier()

    pltpu.sync_copy(acc_sh.at[sl], out_hbm.at[cid].at[sl])


def kernel(x, source, target, edge_type, edge_weights, blocks):
    w8 = jnp.zeros((R, D, D), jnp.float32)
    for b in range(NB):
        s = b * BS
        w8 = w8.at[:, s:s + BS, s:s + BS].set(blocks[:R, b])
    wcat = jnp.transpose(w8, (1, 0, 2)).reshape(D, R * D)

    TN = 1000
    xr = pl.pallas_call(
        _mm_body,
        grid=(N // TN,),
        in_specs=[
            pl.BlockSpec((TN, D), lambda i: (i, 0)),
            pl.BlockSpec((D, R * D), lambda i: (0, 0)),
        ],
        out_specs=pl.BlockSpec((TN, R * D), lambda i: (i, 0)),
        out_shape=jax.ShapeDtypeStruct((N, R * D), jnp.float32),
    )(x, wcat)
    xr = xr.reshape(N * R * 2, DH)

    src_all = jnp.concatenate([source, target])
    tgt_all = jnp.concatenate([target, source])
    et_all = jnp.concatenate([edge_type, edge_type])
    w_all = jnp.concatenate([edge_weights, edge_weights])
    g2 = ((src_all * R + et_all) * 2).astype(jnp.int32)

    pad = EP - E2
    shp = (NS, PH, CHP, K)
    gidx5 = jnp.pad(jnp.stack([g2, g2 + 1]),
                    ((0, 0), (0, pad))).reshape((NC,) + shp)
    tgt4 = jnp.pad(tgt_all.astype(jnp.int32), (0, pad)).reshape(shp)
    w4 = jnp.pad(w_all, (0, pad)).reshape(shp)
    zeros = jnp.zeros((NP, DH), jnp.float32)

    mesh = plsc.VectorSubcoreMesh(core_axis_name="c", subcore_axis_name="s")
    sc_kernel = functools.partial(
        pl.kernel,
        mesh=mesh,
        compiler_params=pltpu.CompilerParams(use_tc_tiling_on_sc=False),
        out_type=jax.ShapeDtypeStruct((NC, NP, DH), jnp.float32),
        scratch_types=[
            pltpu.VMEM((CHP, K), jnp.int32),
            pltpu.VMEM((CHP, K), jnp.int32),
            pltpu.VMEM((CHP, K), jnp.float32),
            pltpu.VMEM((K, DH), jnp.float32),
            pltpu.VMEM((K, DH), jnp.float32),
            pltpu.VMEM_SHARED((NP, DH), jnp.float32),
            pltpu.SemaphoreType.DMA((NSLOT,)),
        ],
    )(_sc_body)
    partials = sc_kernel(xr, gidx5, tgt4, w4, zeros)

    out = pl.pallas_call(
        _interleave_body,
        grid=(N // TN,),
        in_specs=[
            pl.BlockSpec((TN, DH), lambda i: (i, 0)),
            pl.BlockSpec((TN, DH), lambda i: (i, 0)),
        ],
        out_specs=pl.BlockSpec((TN, D), lambda i: (i, 0)),
        out_shape=jax.ShapeDtypeStruct((N, D), jnp.float32),
    )(partials[0, :N], partials[1, :N])
    return out


# speedup vs baseline: 1.9414x; 1.6446x over previous
import functools

import jax
import jax.numpy as jnp
from jax import lax
from jax.experimental import pallas as pl
from jax.experimental.pallas import tpu as pltpu
from jax.experimental.pallas import tpu_sc as plsc

N = 10000
D = 128
DH = D // 2
R = 8
NB = 4
BS = D // NB
E2 = 2 * 320000

NC = 2
NS = 16
L = 16
K = 128
PH = 2
NSLOT = 2
CHP = 158
EP = NS * PH * CHP * K
NP = 10240
ROWS_PER_TILE = NP // NS


def _mm_body(x_ref, w_ref, o_ref):
    o_ref[...] = jnp.dot(x_ref[...], w_ref[...],
                         preferred_element_type=jnp.float32)


def _interleave_body(a_ref, b_ref, o_ref):
    o_ref[:, :DH] = a_ref[...]
    o_ref[:, DH:] = b_ref[...]


def _sc_body(xr_hbm, gidx_hbm, tgt_hbm, w_hbm, zeros_hbm, out_hbm,
             gidx_v, tgt_v, w_v, rows_a, rows_b, acc_sh, gsem):
    cid = lax.axis_index("c")
    sid = lax.axis_index("s")
    bufs = (rows_a, rows_b)

    def gather_start(c, s):
        pltpu.async_copy(xr_hbm.at[gidx_v.at[c]], bufs[s], gsem.at[s])

    def gather_wait(c, s):
        pltpu.make_async_copy(xr_hbm.at[gidx_v.at[c]], bufs[s],
                              gsem.at[s]).wait()

    def scatter_add(c, s):
        pltpu.sync_copy(bufs[s], acc_sh.at[tgt_v.at[c]], add=True)

    def scale(c, s):
        rv = bufs[s]

        @plsc.parallel_loop(0, K // L)
        def _(g):
            wv = w_v[c, pl.ds(g * L, L)]
            for k in range(L):
                wk = wv[k]
                row = g * L + k
                for f in range(DH // L):
                    fs = pl.ds(f * L, L)
                    rv[row, fs] = rv[row, fs] * wk

    sl = pl.ds(sid * ROWS_PER_TILE, ROWS_PER_TILE)
    pltpu.sync_copy(zeros_hbm.at[sl], acc_sh.at[sl])
    plsc.subcore_barrier()

    for ph in range(PH):
        pltpu.sync_copy(gidx_hbm.at[cid].at[sid].at[ph], gidx_v)
        pltpu.sync_copy(tgt_hbm.at[sid].at[ph], tgt_v)
        pltpu.sync_copy(w_hbm.at[sid].at[ph], w_v)

        gather_start(0, 0)

        def pair_body(g, carry):
            c0 = 2 * g
            gather_start(c0 + 1, 1)
            gather_wait(c0, 0)
            scale(c0, 0)
            scatter_add(c0, 0)

            @pl.when(c0 + 2 < CHP)
            def _():
                gather_start(c0 + 2, 0)

            gather_wait(c0 + 1, 1)
            scale(c0 + 1, 1)
            scatter_add(c0 + 1, 1)
            return carry

        lax.fori_loop(0, CHP // 2, pair_body, 0)

    plsc.subcore_barrier()

    pltpu.sync_copy(acc_sh.at[sl], out_hbm.at[cid].at[sl])


def kernel(x, source, target, edge_type, edge_weights, blocks):
    w8 = jnp.zeros((R, D, D), jnp.float32)
    for b in range(NB):
        s = b * BS
        w8 = w8.at[:, s:s + BS, s:s + BS].set(blocks[:R, b])
    wcat = jnp.transpose(w8, (1, 0, 2)).reshape(D, R * D)

    TN = 1000
    xr = pl.pallas_call(
        _mm_body,
        grid=(N // TN,),
        in_specs=[
            pl.BlockSpec((TN, D), lambda i: (i, 0)),
            pl.BlockSpec((D, R * D), lambda i: (0, 0)),
        ],
        out_specs=pl.BlockSpec((TN, R * D), lambda i: (i, 0)),
        out_shape=jax.ShapeDtypeStruct((N, R * D), jnp.float32),
    )(x, wcat)
    xr = xr.reshape(N * R * 2, DH)

    src_all = jnp.concatenate([source, target])
    tgt_all = jnp.concatenate([target, source])
    et_all = jnp.concatenate([edge_type, edge_type])
    w_all = jnp.concatenate([edge_weights, edge_weights])
    g2 = ((src_all * R + et_all) * 2).astype(jnp.int32)

    pad = EP - E2
    shp = (NS, PH, CHP, K)
    gidx5 = jnp.pad(jnp.stack([g2, g2 + 1]),
                    ((0, 0), (0, pad))).reshape((NC,) + shp)
    tgt4 = jnp.pad(tgt_all.astype(jnp.int32), (0, pad)).reshape(shp)
    w4 = jnp.pad(w_all, (0, pad)).reshape(shp)
    zeros = jnp.zeros((NP, DH), jnp.float32)

    mesh = plsc.VectorSubcoreMesh(core_axis_name="c", subcore_axis_name="s")
    sc_kernel = functools.partial(
        pl.kernel,
        mesh=mesh,
        compiler_params=pltpu.CompilerParams(use_tc_tiling_on_sc=False),
        out_type=jax.ShapeDtypeStruct((NC, NP, DH), jnp.float32),
        scratch_types=[
            pltpu.VMEM((CHP, K), jnp.int32),
            pltpu.VMEM((CHP, K), jnp.int32),
            pltpu.VMEM((CHP, K), jnp.float32),
            pltpu.VMEM((K, DH), jnp.float32),
            pltpu.VMEM((K, DH), jnp.float32),
            pltpu.VMEM_SHARED((NP, DH), jnp.float32),
            pltpu.SemaphoreType.DMA((NSLOT,)),
        ],
    )(_sc_body)
    partials = sc_kernel(xr, gidx5, tgt4, w4, zeros)

    out = pl.pallas_call(
        _interleave_body,
        grid=(N // TN,),
        in_specs=[
            pl.BlockSpec((TN, DH), lambda i: (i, 0)),
            pl.BlockSpec((TN, DH), lambda i: (i, 0)),
        ],
        out_specs=pl.BlockSpec((TN, D), lambda i: (i, 0)),
        out_shape=jax.ShapeDtypeStruct((N, D), jnp.float32),
    )(partials[0, :N], partials[1, :N])
    return out
